# Initial kernel scaffold; baseline (speedup 1.0000x reference)
import jax
import jax.numpy as jnp
from jax.experimental import pallas as pl


def kernel(x, edge_index, W_gat, att_src, att_dst, bias_gat, W_sc, b_sc):
    raise NotImplementedError("write your pallas kernel here")


# speedup vs baseline: 17.2436x; 17.2436x over previous
import functools

import jax
import jax.numpy as jnp
from jax import lax
from jax.experimental import pallas as pl
from jax.experimental.pallas import tpu as pltpu
from jax.experimental.pallas import tpu_sc as plsc

N = 10000
E = 320000
C = 128
NPAD = 10240
NB = 10
BLK = 1024
NW = 32
CHUNK = 128
NCHUNK = 79
EPW = CHUNK * NCHUNK
EPAD = NW * EPW
ROWS_PER_SUB = NPAD // 16


def _tc1_body(x_ref, wg_ref, ws_ref, as_ref, ad_ref,
              xw_ref, scx_ref, asrc_ref, adst_ref, es_ref):
    xb = x_ref[...]
    xw = jnp.dot(xb, wg_ref[...], preferred_element_type=jnp.float32)
    xw_ref[...] = xw
    scx_ref[...] = jnp.dot(xb, ws_ref[...], preferred_element_type=jnp.float32)
    a_s = jnp.sum(xw * as_ref[...], axis=1)
    a_d = jnp.sum(xw * ad_ref[...], axis=1)
    asrc_ref[...] = a_s
    adst_ref[...] = a_d
    al = a_s + a_d
    al = jnp.where(al >= 0, al, 0.2 * al)
    es_ref[...] = jnp.exp(al)


def _tc1(x, W_gat, W_sc, att_src_row, att_dst_row):
    return pl.pallas_call(
        _tc1_body,
        grid=(NB,),
        in_specs=[
            pl.BlockSpec((BLK, C), lambda i: (i, 0)),
            pl.BlockSpec((C, C), lambda i: (0, 0)),
            pl.BlockSpec((C, C), lambda i: (0, 0)),
            pl.BlockSpec((1, C), lambda i: (0, 0)),
            pl.BlockSpec((1, C), lambda i: (0, 0)),
        ],
        out_specs=[
            pl.BlockSpec((BLK, C), lambda i: (i, 0)),
            pl.BlockSpec((BLK, C), lambda i: (i, 0)),
            pl.BlockSpec((BLK,), lambda i: (i,)),
            pl.BlockSpec((BLK,), lambda i: (i,)),
            pl.BlockSpec((BLK,), lambda i: (i,)),
        ],
        out_shape=[
            jax.ShapeDtypeStruct((N, C), jnp.float32),
            jax.ShapeDtypeStruct((N, C), jnp.float32),
            jax.ShapeDtypeStruct((N,), jnp.float32),
            jax.ShapeDtypeStruct((N,), jnp.float32),
            jax.ShapeDtypeStruct((N,), jnp.float32),
        ],
    )(x, W_gat, W_sc, att_src_row, att_dst_row)


def _sc_body(src_hbm, dst_hbm, asrc_hbm, adst_hbm, xw_hbm,
             out_hbm, den_hbm,
             rows_v, src_v, dst_v, e_v, av_v, ad_v,
             out_sh, den_sh, sem):
    cid = lax.axis_index("c")
    sid = lax.axis_index("s")
    wid = sid * 2 + cid
    base = wid * EPW

    def _zrow(r, carry):
        for b in range(8):
            rows_v[r, pl.ds(b * 16, 16)] = jnp.zeros((16,), jnp.float32)
        return carry
    lax.fori_loop(0, CHUNK, _zrow, 0)
    for g in range(8):
        e_v[pl.ds(g * 16, 16)] = jnp.zeros((16,), jnp.float32)

    zbase = sid * ROWS_PER_SUB
    for q in range(ROWS_PER_SUB // CHUNK):
        pltpu.sync_copy(rows_v, out_sh.at[pl.ds(zbase + q * CHUNK, CHUNK)])
        pltpu.sync_copy(e_v, den_sh.at[pl.ds(zbase + q * CHUNK, CHUNK)])
    plsc.subcore_barrier()

    def _chunk(ci, carry):
        ebase = base + ci * CHUNK
        pltpu.sync_copy(src_hbm.at[pl.ds(ebase, CHUNK)], src_v)
        pltpu.sync_copy(dst_hbm.at[pl.ds(ebase, CHUNK)], dst_v)
        pltpu.async_copy(asrc_hbm.at[src_v], av_v, sem).wait()
        pltpu.async_copy(adst_hbm.at[dst_v], ad_v, sem).wait()
        pltpu.async_copy(xw_hbm.at[src_v], rows_v, sem).wait()
        for g in range(8):
            al = av_v[pl.ds(g * 16, 16)] + ad_v[pl.ds(g * 16, 16)]
            al = jnp.where(al >= 0, al, 0.2 * al)
            e_v[pl.ds(g * 16, 16)] = jnp.exp(al)

        def _rowgrp(g, c2):
            r0 = g * 16
            ev16 = e_v[pl.ds(r0, 16)]
            for l in range(16):
                ev = ev16[l]
                r = r0 + l
                for b in range(8):
                    rows_v[r, pl.ds(b * 16, 16)] = (
                        rows_v[r, pl.ds(b * 16, 16)] * ev)
            return c2
        lax.fori_loop(0, CHUNK // 16, _rowgrp, 0)

        pltpu.sync_copy(rows_v, out_sh.at[dst_v], add=True)
        pltpu.sync_copy(e_v, den_sh.at[dst_v], add=True)
        return carry
    lax.fori_loop(0, NCHUNK, _chunk, 0)

    plsc.subcore_barrier()
    pltpu.sync_copy(out_sh.at[pl.ds(zbase, ROWS_PER_SUB)],
                    out_hbm.at[cid, pl.ds(zbase, ROWS_PER_SUB)])
    pltpu.sync_copy(den_sh.at[pl.ds(zbase, ROWS_PER_SUB)],
                    den_hbm.at[cid, 0, pl.ds(zbase, ROWS_PER_SUB)])


_sc_edge_pass = functools.partial(
    pl.kernel,
    out_type=(jax.ShapeDtypeStruct((2, NPAD, C), jnp.float32),
              jax.ShapeDtypeStruct((2, 1, NPAD), jnp.float32)),
    mesh=plsc.VectorSubcoreMesh(core_axis_name="c", subcore_axis_name="s"),
    scratch_types=[
        pltpu.VMEM((CHUNK, C), jnp.float32),
        pltpu.VMEM((CHUNK,), jnp.int32),
        pltpu.VMEM((CHUNK,), jnp.int32),
        pltpu.VMEM((CHUNK,), jnp.float32),
        pltpu.VMEM((CHUNK,), jnp.float32),
        pltpu.VMEM((CHUNK,), jnp.float32),
        pltpu.VMEM_SHARED((NPAD, C), jnp.float32),
        pltpu.VMEM_SHARED((NPAD,), jnp.float32),
        pltpu.SemaphoreType.DMA,
    ],
)(_sc_body)


def _tc2_body(o_ref, d_ref, es_ref, xw_ref, scx_ref, b_ref, bs_ref, gx_ref):
    es = es_ref[...]
    den = d_ref[0, :] + d_ref[1, :] + es
    num = o_ref[0] + o_ref[1] + es[:, None] * xw_ref[...]
    gx_ref[...] = (num / den[:, None] + b_ref[...]
                   + scx_ref[...] + bs_ref[...])


def _tc2(out_p, den_p, es, xw, scx, bias_row, bsc_row):
    return pl.pallas_call(
        _tc2_body,
        grid=(NB,),
        in_specs=[
            pl.BlockSpec((2, BLK, C), lambda i: (0, i, 0)),
            pl.BlockSpec((2, BLK), lambda i: (0, i)),
            pl.BlockSpec((BLK,), lambda i: (i,)),
            pl.BlockSpec((BLK, C), lambda i: (i, 0)),
            pl.BlockSpec((BLK, C), lambda i: (i, 0)),
            pl.BlockSpec((1, C), lambda i: (0, 0)),
            pl.BlockSpec((1, C), lambda i: (0, 0)),
        ],
        out_specs=pl.BlockSpec((BLK, C), lambda i: (i, 0)),
        out_shape=jax.ShapeDtypeStruct((N, C), jnp.float32),
    )(out_p, den_p, es, xw, scx, bias_row, bsc_row)


def kernel(x, edge_index, W_gat, att_src, att_dst, bias_gat, W_sc, b_sc):
    att_s = att_src.reshape(1, C)
    att_d = att_dst.reshape(1, C)
    xw, scx, asrc, adst, es = _tc1(x, W_gat, W_sc, att_s, att_d)

    asrc_pad = jnp.pad(asrc, (0, NPAD - N))
    adst_pad = jnp.pad(adst, (0, NPAD - N))
    src_pad = jnp.concatenate(
        [edge_index[0], jnp.zeros((EPAD - E,), jnp.int32)])
    dst_pad = jnp.concatenate(
        [edge_index[1], jnp.full((EPAD - E,), N + 8, jnp.int32)])

    out_p, den_p = _sc_edge_pass(src_pad, dst_pad, asrc_pad, adst_pad, xw)

    den_p = den_p.reshape(2, NPAD)
    gx = _tc2(out_p, den_p, es, xw, scx,
              bias_gat.reshape(1, C), b_sc.reshape(1, C))
    return (gx, edge_index)


# speedup vs baseline: 18.8661x; 1.0941x over previous
import functools

import jax
import jax.numpy as jnp
from jax import lax
from jax.experimental import pallas as pl
from jax.experimental.pallas import tpu as pltpu
from jax.experimental.pallas import tpu_sc as plsc

N = 10000
E = 320000
C = 128
NPAD = 10240
NB = 10
BLK = 1024
NW = 32
CHUNK = 128
NCHUNK = 80
EPW = CHUNK * NCHUNK
EPAD = NW * EPW
ROWS_PER_SUB = NPAD // 16


def _tc1_body(x_ref, wg_ref, ws_ref, as_ref, ad_ref,
              xw_ref, scx_ref, asrc_ref, adst_ref, es_ref):
    xb = x_ref[...]
    xw = jnp.dot(xb, wg_ref[...], preferred_element_type=jnp.float32)
    xw_ref[...] = xw
    scx_ref[...] = jnp.dot(xb, ws_ref[...], preferred_element_type=jnp.float32)
    a_s = jnp.sum(xw * as_ref[...], axis=1)
    a_d = jnp.sum(xw * ad_ref[...], axis=1)
    asrc_ref[...] = a_s
    adst_ref[...] = a_d
    al = a_s + a_d
    al = jnp.where(al >= 0, al, 0.2 * al)
    es_ref[...] = jnp.exp(al)


def _tc1(x, W_gat, W_sc, att_src_row, att_dst_row):
    return pl.pallas_call(
        _tc1_body,
        grid=(NB,),
        in_specs=[
            pl.BlockSpec((BLK, C), lambda i: (i, 0)),
            pl.BlockSpec((C, C), lambda i: (0, 0)),
            pl.BlockSpec((C, C), lambda i: (0, 0)),
            pl.BlockSpec((1, C), lambda i: (0, 0)),
            pl.BlockSpec((1, C), lambda i: (0, 0)),
        ],
        out_specs=[
            pl.BlockSpec((BLK, C), lambda i: (i, 0)),
            pl.BlockSpec((BLK, C), lambda i: (i, 0)),
            pl.BlockSpec((BLK,), lambda i: (i,)),
            pl.BlockSpec((BLK,), lambda i: (i,)),
            pl.BlockSpec((BLK,), lambda i: (i,)),
        ],
        out_shape=[
            jax.ShapeDtypeStruct((N, C), jnp.float32),
            jax.ShapeDtypeStruct((N, C), jnp.float32),
            jax.ShapeDtypeStruct((N,), jnp.float32),
            jax.ShapeDtypeStruct((N,), jnp.float32),
            jax.ShapeDtypeStruct((N,), jnp.float32),
        ],
    )(x, W_gat, W_sc, att_src_row, att_dst_row)


def _sc_body(src_hbm, dst_hbm, asrc_hbm, adst_hbm, xw_hbm,
             out_hbm, den_hbm,
             rows0, rows1, src0, src1, dst0, dst1, e0, e1,
             av0, av1, ad0, ad1,
             si0, si1, di0, di1, sr0, sr1, sa0, sa1, sb0, sb1,
             wr0, wr1, we0, we1,
             out_sh, den_sh):
    rowsb = (rows0, rows1)
    srcb = (src0, src1)
    dstb = (dst0, dst1)
    eb = (e0, e1)
    avb = (av0, av1)
    adb = (ad0, ad1)
    sem_si = (si0, si1)
    sem_di = (di0, di1)
    sem_r = (sr0, sr1)
    sem_a = (sa0, sa1)
    sem_b = (sb0, sb1)
    sem_wr = (wr0, wr1)
    sem_we = (we0, we1)

    cid = lax.axis_index("c")
    sid = lax.axis_index("s")
    wid = sid * 2 + cid
    base = wid * EPW

    def _zrow(r, carry):
        for b in range(8):
            rows0[r, pl.ds(b * 16, 16)] = jnp.zeros((16,), jnp.float32)
        return carry
    lax.fori_loop(0, CHUNK, _zrow, 0)
    for g in range(8):
        e0[pl.ds(g * 16, 16)] = jnp.zeros((16,), jnp.float32)

    zbase = sid * ROWS_PER_SUB
    for q in range(ROWS_PER_SUB // CHUNK):
        pltpu.sync_copy(rows0, out_sh.at[pl.ds(zbase + q * CHUNK, CHUNK)])
        pltpu.sync_copy(e0, den_sh.at[pl.ds(zbase + q * CHUNK, CHUNK)])
    plsc.subcore_barrier()

    def issue_idx(k, p):
        ebase = base + k * CHUNK
        pltpu.async_copy(src_hbm.at[pl.ds(ebase, CHUNK)], srcb[p], sem_si[p])
        pltpu.async_copy(dst_hbm.at[pl.ds(ebase, CHUNK)], dstb[p], sem_di[p])

    def wait_idx(p):
        pltpu.make_async_copy(src_hbm.at[pl.ds(0, CHUNK)], srcb[p],
                              sem_si[p]).wait()
        pltpu.make_async_copy(dst_hbm.at[pl.ds(0, CHUNK)], dstb[p],
                              sem_di[p]).wait()

    def issue_gathers(p):
        pltpu.async_copy(asrc_hbm.at[srcb[p]], avb[p], sem_a[p])
        pltpu.async_copy(adst_hbm.at[dstb[p]], adb[p], sem_b[p])
        pltpu.async_copy(xw_hbm.at[srcb[p]], rowsb[p], sem_r[p])

    def wait_gathers(p):
        pltpu.make_async_copy(asrc_hbm.at[srcb[p]], avb[p], sem_a[p]).wait()
        pltpu.make_async_copy(adst_hbm.at[dstb[p]], adb[p], sem_b[p]).wait()
        pltpu.make_async_copy(xw_hbm.at[srcb[p]], rowsb[p], sem_r[p]).wait()

    def issue_scatters(p):
        pltpu.async_copy(rowsb[p], out_sh.at[dstb[p]], sem_wr[p], add=True)
        pltpu.async_copy(eb[p], den_sh.at[dstb[p]], sem_we[p], add=True)

    def wait_scatters(p):
        pltpu.make_async_copy(rowsb[p], out_sh.at[dstb[p]], sem_wr[p]).wait()
        pltpu.make_async_copy(eb[p], den_sh.at[dstb[p]], sem_we[p]).wait()

    def compute_scale(p):
        rows_v, e_v, av_v, ad_v = rowsb[p], eb[p], avb[p], adb[p]
        for g in range(8):
            al = av_v[pl.ds(g * 16, 16)] + ad_v[pl.ds(g * 16, 16)]
            al = jnp.where(al >= 0, al, 0.2 * al)
            e_v[pl.ds(g * 16, 16)] = jnp.exp(al)

        def _rowgrp(g, c2):
            r0 = g * 16
            ev16 = e_v[pl.ds(r0, 16)]
            for l in range(16):
                ev = ev16[l]
                r = r0 + l
                for b in range(8):
                    rows_v[r, pl.ds(b * 16, 16)] = (
                        rows_v[r, pl.ds(b * 16, 16)] * ev)
            return c2
        lax.fori_loop(0, CHUNK // 16, _rowgrp, 0)

    def phase(k, p):
        @pl.when(k >= 1)
        def _():
            wait_scatters(1 - p)

        @pl.when(k + 1 < NCHUNK)
        def _():
            issue_idx(k + 1, 1 - p)
            wait_idx(1 - p)
            issue_gathers(1 - p)
        wait_gathers(p)
        compute_scale(p)
        issue_scatters(p)

    issue_idx(0, 0)
    wait_idx(0)
    issue_gathers(0)

    def _outer(t, carry):
        phase(2 * t, 0)
        phase(2 * t + 1, 1)
        return carry
    lax.fori_loop(0, NCHUNK // 2, _outer, 0)
    wait_scatters(1)

    plsc.subcore_barrier()
    pltpu.sync_copy(out_sh.at[pl.ds(zbase, ROWS_PER_SUB)],
                    out_hbm.at[cid, pl.ds(zbase, ROWS_PER_SUB)])
    pltpu.sync_copy(den_sh.at[pl.ds(zbase, ROWS_PER_SUB)],
                    den_hbm.at[cid, 0, pl.ds(zbase, ROWS_PER_SUB)])


_sc_edge_pass = functools.partial(
    pl.kernel,
    out_type=(jax.ShapeDtypeStruct((2, NPAD, C), jnp.float32),
              jax.ShapeDtypeStruct((2, 1, NPAD), jnp.float32)),
    mesh=plsc.VectorSubcoreMesh(core_axis_name="c", subcore_axis_name="s"),
    scratch_types=[
        pltpu.VMEM((CHUNK, C), jnp.float32),
        pltpu.VMEM((CHUNK, C), jnp.float32),
        pltpu.VMEM((CHUNK,), jnp.int32),
        pltpu.VMEM((CHUNK,), jnp.int32),
        pltpu.VMEM((CHUNK,), jnp.int32),
        pltpu.VMEM((CHUNK,), jnp.int32),
        pltpu.VMEM((CHUNK,), jnp.float32),
        pltpu.VMEM((CHUNK,), jnp.float32),
        pltpu.VMEM((CHUNK,), jnp.float32),
        pltpu.VMEM((CHUNK,), jnp.float32),
        pltpu.VMEM((CHUNK,), jnp.float32),
        pltpu.VMEM((CHUNK,), jnp.float32),
    ] + [pltpu.SemaphoreType.DMA] * 14 + [
        pltpu.VMEM_SHARED((NPAD, C), jnp.float32),
        pltpu.VMEM_SHARED((NPAD,), jnp.float32),
    ],
)(_sc_body)


def _tc2_body(o_ref, d_ref, es_ref, xw_ref, scx_ref, b_ref, bs_ref, gx_ref):
    es = es_ref[...]
    den = d_ref[0, :] + d_ref[1, :] + es
    num = o_ref[0] + o_ref[1] + es[:, None] * xw_ref[...]
    gx_ref[...] = (num / den[:, None] + b_ref[...]
                   + scx_ref[...] + bs_ref[...])


def _tc2(out_p, den_p, es, xw, scx, bias_row, bsc_row):
    return pl.pallas_call(
        _tc2_body,
        grid=(NB,),
        in_specs=[
            pl.BlockSpec((2, BLK, C), lambda i: (0, i, 0)),
            pl.BlockSpec((2, BLK), lambda i: (0, i)),
            pl.BlockSpec((BLK,), lambda i: (i,)),
            pl.BlockSpec((BLK, C), lambda i: (i, 0)),
            pl.BlockSpec((BLK, C), lambda i: (i, 0)),
            pl.BlockSpec((1, C), lambda i: (0, 0)),
            pl.BlockSpec((1, C), lambda i: (0, 0)),
        ],
        out_specs=pl.BlockSpec((BLK, C), lambda i: (i, 0)),
        out_shape=jax.ShapeDtypeStruct((N, C), jnp.float32),
    )(out_p, den_p, es, xw, scx, bias_row, bsc_row)


def kernel(x, edge_index, W_gat, att_src, att_dst, bias_gat, W_sc, b_sc):
    att_s = att_src.reshape(1, C)
    att_d = att_dst.reshape(1, C)
    xw, scx, asrc, adst, es = _tc1(x, W_gat, W_sc, att_s, att_d)

    asrc_pad = jnp.pad(asrc, (0, NPAD - N))
    adst_pad = jnp.pad(adst, (0, NPAD - N))
    src_pad = jnp.concatenate(
        [edge_index[0], jnp.zeros((EPAD - E,), jnp.int32)])
    dst_pad = jnp.concatenate(
        [edge_index[1], jnp.full((EPAD - E,), N + 8, jnp.int32)])

    out_p, den_p = _sc_edge_pass(src_pad, dst_pad, asrc_pad, adst_pad, xw)

    den_p = den_p.reshape(2, NPAD)
    gx = _tc2(out_p, den_p, es, xw, scx,
              bias_gat.reshape(1, C), b_sc.reshape(1, C))
    return (gx, edge_index)
